# baseline (speedup 1.0000x reference)
import functools

import jax
import jax.numpy as jnp
from jax import lax
from jax.experimental import pallas as pl
from jax.experimental.pallas import tpu as pltpu
from jax.experimental.pallas import tpu_sc as plsc

N = 10000
E = 320000
F = 128
H = 64
K = 32

NC = 2
NS = 16
LANES = 16

NPAD = 10240
ROWS_PER_SUB = NPAD // NS
BATCH = 128
EPAD = 327680
TCHUNKS = EPAD // (NS * BATCH)
EREAL = E // BATCH
LAST_REAL = EREAL - (NS - 1) * TCHUNKS
PAD_ROWS = TCHUNKS - LAST_REAL

NBUF = 5
PDIST = 2
OUTER = TCHUNKS // NBUF

RB = 5120


def _maxk_mask2(h0, h1):
    b0 = lax.bitcast_convert_type(h0, jnp.int32)
    b1 = lax.bitcast_convert_type(h1, jnp.int32)
    k0 = jnp.where(b0 >= 0, b0, b0 ^ jnp.int32(0x7FFFFFFF))
    k1 = jnp.where(b1 >= 0, b1, b1 ^ jnp.int32(0x7FFFFFFF))
    hi0 = (k0 >> 16).astype(jnp.int16)
    hi1 = (k1 >> 16).astype(jnp.int16)
    lw0 = ((k0 & jnp.int32(0xFFFF)) ^ jnp.int32(0x8000)).astype(jnp.int16)
    lw1 = ((k1 & jnp.int32(0xFFFF)) ^ jnp.int32(0x8000)).astype(jnp.int16)

    one = jnp.bfloat16(1.0)
    zero = jnp.bfloat16(0.0)
    ones_col = jnp.full((H, 1), 1.0, jnp.bfloat16)
    kkf = jnp.float32(K)
    zcol = jnp.sum(jnp.zeros_like(h0), axis=-1, keepdims=True).astype(
        jnp.int32)

    def search(count_fn):
        lo = zcol + jnp.int32(-32768)
        hi = zcol + jnp.int32(32768)

        def body(_, carry):
            lo, hi = carry
            mid = (lo + hi) >> 1
            p = count_fn(mid.astype(jnp.int16)) >= kkf
            return jnp.where(p, mid, lo), jnp.where(p, hi, mid)

        lo, hi = lax.fori_loop(0, 16, body, (lo, hi))
        return lo

    def cnt_hi(m):
        i0 = jnp.where(hi0 >= m, one, zero)
        i1 = jnp.where(hi1 >= m, one, zero)
        return _dot(i0, ones_col) + _dot(i1, ones_col)

    t16 = search(cnt_hi).astype(jnp.int16)
    strict0 = hi0 > t16
    strict1 = hi1 > t16
    buck0 = hi0 == t16
    buck1 = hi1 == t16
    c_hi = (_dot(jnp.where(strict0, one, zero), ones_col)
            + _dot(jnp.where(strict1, one, zero), ones_col))

    def cnt_low(m):
        i0 = jnp.where(buck0 & (lw0 >= m), one, zero)
        i1 = jnp.where(buck1 & (lw1 >= m), one, zero)
        return c_hi + _dot(i0, ones_col) + _dot(i1, ones_col)

    tlow = search(cnt_low).astype(jnp.int16)
    m0 = strict0 | (buck0 & (lw0 >= tlow))
    m1 = strict1 | (buck1 & (lw1 >= tlow))
    z = jnp.float32(0.0)
    return jnp.where(m0, h0, z), jnp.where(m1, h1, z)


def _dot(a, b):
    return jnp.dot(a, b, preferred_element_type=jnp.float32)


def _in_maxk_body(x_ref, wt0_ref, wt1_ref, b0_ref, b1_ref, o0_ref, o1_ref):
    h0 = _dot(x_ref[...], wt0_ref[...]) + b0_ref[...]
    h1 = _dot(x_ref[...], wt1_ref[...]) + b1_ref[...]
    o0_ref[...], o1_ref[...] = _maxk_mask2(h0, h1)


def _self_mm_body(m0_ref, m1_ref, wsaa, wsab, wsba, wsbb, b0_ref, b1_ref,
                  o0_ref, o1_ref):
    m0, m1 = m0_ref[...], m1_ref[...]
    o0_ref[...] = _dot(m0, wsaa[...]) + _dot(m1, wsba[...]) + b0_ref[...]
    o1_ref[...] = _dot(m0, wsab[...]) + _dot(m1, wsbb[...]) + b1_ref[...]


def _neigh_halves(hs0, hs1, a0, a1, inv, wn):
    hn0 = a0 * inv
    hn1 = a1 * inv
    h0 = hs0 + _dot(hn0, wn[0][0]) + _dot(hn1, wn[1][0])
    h1 = hs1 + _dot(hn0, wn[0][1]) + _dot(hn1, wn[1][1])
    return h0, h1


def _sage_maxk_body(hs0_ref, hs1_ref, a0_ref, a1_ref, deg_ref,
                    wnaa, wnab, wnba, wnbb, o0_ref, o1_ref):
    inv = jnp.float32(1.0) / jnp.maximum(deg_ref[...], jnp.float32(1.0))
    h0, h1 = _neigh_halves(
        hs0_ref[...], hs1_ref[...], a0_ref[...], a1_ref[...], inv,
        ((wnaa[...], wnab[...]), (wnba[...], wnbb[...])))
    o0_ref[...], o1_ref[...] = _maxk_mask2(h0, h1)


def _sage_out_body(hs0_ref, hs1_ref, a0_ref, a1_ref, deg_ref,
                   wnaa, wnab, wnba, wnbb, woa_ref, wob_ref, bo_ref, o_ref):
    inv = jnp.float32(1.0) / jnp.maximum(deg_ref[...], jnp.float32(1.0))
    h0, h1 = _neigh_halves(
        hs0_ref[...], hs1_ref[...], a0_ref[...], a1_ref[...], inv,
        ((wnaa[...], wnab[...]), (wnba[...], wnbb[...])))
    o_ref[...] = (_dot(h0, woa_ref[...]) + _dot(h1, wob_ref[...])
                  + bo_ref[...])


def _row_spec(rb, w):
    return pl.BlockSpec((rb, w), lambda i: (i, 0))


def _full_spec(shape):
    return pl.BlockSpec(shape, lambda i: (0, 0))


def _half_out(rb, nrows):
    return (
        [jax.ShapeDtypeStruct((nrows, H), jnp.float32)] * 2,
        [_row_spec(rb, H)] * 2,
    )


_in_maxk = pl.pallas_call(
    _in_maxk_body,
    grid=(NPAD // RB,),
    in_specs=[_row_spec(RB, F), _full_spec((F, H)), _full_spec((F, H)),
              _full_spec((1, H)), _full_spec((1, H))],
    out_specs=_half_out(RB, NPAD)[1],
    out_shape=_half_out(RB, NPAD)[0],
)

_QSPECS = [_full_spec((H, H))] * 4

_self_mm = pl.pallas_call(
    _self_mm_body,
    grid=(NPAD // RB,),
    in_specs=([_row_spec(RB, H)] * 2 + _QSPECS + [_full_spec((1, H))] * 2),
    out_specs=_half_out(RB, NPAD)[1],
    out_shape=_half_out(RB, NPAD)[0],
)

_sage_maxk = pl.pallas_call(
    _sage_maxk_body,
    grid=(NPAD // RB,),
    in_specs=([_row_spec(RB, H)] * 4
              + [pl.BlockSpec((RB, 1), lambda i: (i, 0))]
              + _QSPECS),
    out_specs=_half_out(RB, NPAD)[1],
    out_shape=_half_out(RB, NPAD)[0],
)

_RB_OUT = 2000

_sage_out = pl.pallas_call(
    _sage_out_body,
    grid=(N // _RB_OUT,),
    in_specs=([_row_spec(_RB_OUT, H)] * 4
              + [pl.BlockSpec((_RB_OUT, 1), lambda i: (i, 0))]
              + _QSPECS
              + [_full_spec((H, F))] * 2 + [_full_spec((1, F))]),
    out_specs=_row_spec(_RB_OUT, F),
    out_shape=jax.ShapeDtypeStruct((N, F), jnp.float32),
)


def _quarters(w):
    return (w[:H, :H], w[:H, H:], w[H:, :H], w[H:, H:])


def _make_sc_agg(with_deg):
    mesh = plsc.VectorSubcoreMesh(core_axis_name="c", subcore_axis_name="s")
    out_types = [jax.ShapeDtypeStruct((NC, NPAD, H), jnp.float32)]
    scratch = [
        pltpu.VMEM((TCHUNKS, BATCH), jnp.int32),
        pltpu.VMEM((TCHUNKS, BATCH), jnp.int32),
        pltpu.VMEM((NBUF, BATCH, H), jnp.float32),
        pltpu.VMEM_SHARED((NPAD, H), jnp.float32),
    ]
    scratch += [pltpu.SemaphoreType.DMA] * (2 * NBUF)
    if with_deg:
        out_types.append(jax.ShapeDtypeStruct((NC, NPAD), jnp.float32))
        scratch += [
            pltpu.VMEM((BATCH,), jnp.float32),
            pltpu.VMEM_SHARED((NPAD,), jnp.float32),
            pltpu.SemaphoreType.DMA,
        ]

    def body(mh0_hbm, mh1_hbm, src_hbm, dst_hbm, sp_hbm, dp_hbm,
             z2_hbm, z1_hbm, *rest):
        if with_deg:
            agg_out, deg_out = rest[0], rest[1]
            rest = rest[2:]
        else:
            agg_out = rest[0]
            rest = rest[1:]
        src_v, dst_v, rows_v, agg_sh = rest[0], rest[1], rest[2], rest[3]
        gsem = rest[4:4 + NBUF]
        ssem = rest[4 + NBUF:4 + 2 * NBUF]
        if with_deg:
            ones_v, deg_sh, dsem = rest[4 + 2 * NBUF:]
        cid = lax.axis_index("c")
        sid = lax.axis_index("s")
        row0 = sid * ROWS_PER_SUB

        pltpu.sync_copy(z2_hbm.at[pl.ds(row0, ROWS_PER_SUB)],
                        agg_sh.at[pl.ds(row0, ROWS_PER_SUB)])
        if with_deg:
            pltpu.sync_copy(z1_hbm.at[pl.ds(row0, ROWS_PER_SUB)],
                            deg_sh.at[pl.ds(row0, ROWS_PER_SUB)])

            def fill(i, c):
                ones_v[pl.ds(i * LANES, LANES)] = jnp.full((LANES,), 1.0,
                                                           jnp.float32)
                return c
            lax.fori_loop(0, BATCH // LANES, fill, 0)

        @pl.when(sid < NS - 1)
        def _():
            pltpu.sync_copy(src_hbm.at[pl.ds(sid * TCHUNKS, TCHUNKS)], src_v)
            pltpu.sync_copy(dst_hbm.at[pl.ds(sid * TCHUNKS, TCHUNKS)], dst_v)

        @pl.when(sid == NS - 1)
        def _():
            pltpu.sync_copy(src_hbm.at[pl.ds((NS - 1) * TCHUNKS, LAST_REAL)],
                            src_v.at[pl.ds(0, LAST_REAL)])
            pltpu.sync_copy(sp_hbm, src_v.at[pl.ds(LAST_REAL, PAD_ROWS)])
            pltpu.sync_copy(dst_hbm.at[pl.ds((NS - 1) * TCHUNKS, LAST_REAL)],
                            dst_v.at[pl.ds(0, LAST_REAL)])
            pltpu.sync_copy(dp_hbm, dst_v.at[pl.ds(LAST_REAL, PAD_ROWS)])
        plsc.subcore_barrier()

        if with_deg:
            def deg_fire(g, c):
                pltpu.async_copy(ones_v, deg_sh.at[dst_v.at[g]], dsem,
                                 add=True)
                return c
            lax.fori_loop(0, TCHUNKS, deg_fire, 0)

        def run(table_hbm):
            def g_start(p, bp):
                pltpu.async_copy(table_hbm.at[src_v.at[p]], rows_v.at[bp],
                                 gsem[bp])

            def g_wait(g, b):
                pltpu.make_async_copy(table_hbm.at[src_v.at[g]], rows_v.at[b],
                                      gsem[b]).wait()

            def s_start(g, b):
                pltpu.async_copy(rows_v.at[b], agg_sh.at[dst_v.at[g]],
                                 ssem[b], add=True)

            def s_wait(g, b):
                pltpu.make_async_copy(rows_v.at[b], agg_sh.at[dst_v.at[g]],
                                      ssem[b]).wait()

            for b in range(PDIST):
                g_start(b, b)

            def outer(t, c):
                g0 = t * NBUF
                for b in range(NBUF):
                    g = g0 + b
                    g_wait(g, b)
                    s_start(g, b)
                    p = g + PDIST
                    bp = (b + PDIST) % NBUF

                    @pl.when(jnp.logical_and(p >= NBUF, p < TCHUNKS))
                    def _():
                        s_wait(p - NBUF, bp)

                    @pl.when(p < TCHUNKS)
                    def _():
                        g_start(p, bp)
                return c
            lax.fori_loop(0, OUTER, outer, 0)

            for b in range(NBUF):
                s_wait(TCHUNKS - NBUF + b, b)

        @pl.when(cid == 0)
        def _():
            run(mh0_hbm)

        @pl.when(cid == 1)
        def _():
            run(mh1_hbm)

        if with_deg:
            def deg_drain(g, c):
                pltpu.make_async_copy(ones_v, deg_sh.at[dst_v.at[g]],
                                      dsem).wait()
                return c
            lax.fori_loop(0, TCHUNKS, deg_drain, 0)

        plsc.subcore_barrier()
        pltpu.sync_copy(agg_sh.at[pl.ds(row0, ROWS_PER_SUB)],
                        agg_out.at[cid, pl.ds(row0, ROWS_PER_SUB)])
        if with_deg:
            pltpu.sync_copy(deg_sh.at[pl.ds(row0, ROWS_PER_SUB)],
                            deg_out.at[cid, pl.ds(row0, ROWS_PER_SUB)])

    return pl.kernel(body, out_type=tuple(out_types), mesh=mesh,
                     scratch_types=scratch,
                     compiler_params=pltpu.CompilerParams(
                         use_tc_tiling_on_sc=False))


@functools.lru_cache(maxsize=None)
def _get_sc_agg(with_deg):
    return _make_sc_agg(with_deg)


def kernel(x, edge_index, W_in, b_in, W_self_0, W_neigh_0, bias_0,
           W_self_1, W_neigh_1, bias_1, W_out, b_out):
    src_p = edge_index[0].reshape(EREAL, BATCH)
    dst_p = edge_index[1].reshape(EREAL, BATCH)
    sp = jnp.zeros((PAD_ROWS, BATCH), jnp.int32)
    dp = jnp.full((PAD_ROWS, BATCH), NPAD - 1, jnp.int32)
    x_p = jnp.concatenate([x, jnp.zeros((NPAD - N, F), jnp.float32)], axis=0)
    z2 = jnp.zeros((NPAD, H), jnp.float32)
    z1 = jnp.zeros((NPAD,), jnp.float32)

    wt_in = W_in.T
    ws0, wn0, ws1, wn1 = W_self_0.T, W_neigh_0.T, W_self_1.T, W_neigh_1.T

    m0a, m0b = _in_maxk(x_p, wt_in[:, :H], wt_in[:, H:],
                        b_in[:H].reshape(1, H), b_in[H:].reshape(1, H))
    (agg0, deg) = _get_sc_agg(True)(m0a, m0b, src_p, dst_p, sp, dp, z2, z1)
    degs = deg[0].reshape(NPAD, 1)
    hs0a, hs0b = _self_mm(m0a, m0b, *_quarters(ws0),
                          bias_0[:H].reshape(1, H), bias_0[H:].reshape(1, H))
    m1a, m1b = _sage_maxk(hs0a, hs0b, agg0[0], agg0[1], degs,
                          *_quarters(wn0))
    (agg1,) = _get_sc_agg(False)(m1a, m1b, src_p, dst_p, sp, dp, z2, z1)
    hs1a, hs1b = _self_mm(m1a, m1b, *_quarters(ws1),
                          bias_1[:H].reshape(1, H), bias_1[H:].reshape(1, H))
    out = _sage_out(hs1a, hs1b, agg1[0], agg1[1], degs,
                    *_quarters(wn1),
                    W_out.T[:H], W_out.T[H:], b_out.reshape(1, F))
    return out


# speedup vs baseline: 1.0206x; 1.0206x over previous
import functools

import jax
import jax.numpy as jnp
from jax import lax
from jax.experimental import pallas as pl
from jax.experimental.pallas import tpu as pltpu
from jax.experimental.pallas import tpu_sc as plsc

N = 10000
E = 320000
F = 128
H = 64
K = 32

NC = 2
NS = 16
LANES = 16

NPAD = 10240
ROWS_PER_SUB = NPAD // NS
BATCH = 128
EPAD = 327680
TCHUNKS = EPAD // (NS * BATCH)
EREAL = E // BATCH
LAST_REAL = EREAL - (NS - 1) * TCHUNKS
PAD_ROWS = TCHUNKS - LAST_REAL

NBUF = 5
PDIST = 3
OUTER = TCHUNKS // NBUF

RB = 5120


def _maxk_mask2(h0, h1):
    b0 = lax.bitcast_convert_type(h0, jnp.int32)
    b1 = lax.bitcast_convert_type(h1, jnp.int32)
    k0 = jnp.where(b0 >= 0, b0, b0 ^ jnp.int32(0x7FFFFFFF))
    k1 = jnp.where(b1 >= 0, b1, b1 ^ jnp.int32(0x7FFFFFFF))
    hi0 = (k0 >> 16).astype(jnp.int16)
    hi1 = (k1 >> 16).astype(jnp.int16)
    lw0 = ((k0 & jnp.int32(0xFFFF)) ^ jnp.int32(0x8000)).astype(jnp.int16)
    lw1 = ((k1 & jnp.int32(0xFFFF)) ^ jnp.int32(0x8000)).astype(jnp.int16)

    one = jnp.bfloat16(1.0)
    zero = jnp.bfloat16(0.0)
    ones_col = jnp.full((H, 1), 1.0, jnp.bfloat16)
    kkf = jnp.float32(K)
    zcol = jnp.sum(jnp.zeros_like(h0), axis=-1, keepdims=True).astype(
        jnp.int32)

    def search(count_fn):
        lo = zcol + jnp.int32(-32768)
        hi = zcol + jnp.int32(32768)

        def body(_, carry):
            lo, hi = carry
            mid = (lo + hi) >> 1
            p = count_fn(mid.astype(jnp.int16)) >= kkf
            return jnp.where(p, mid, lo), jnp.where(p, hi, mid)

        lo, hi = lax.fori_loop(0, 16, body, (lo, hi))
        return lo

    def cnt_hi(m):
        i0 = jnp.where(hi0 >= m, one, zero)
        i1 = jnp.where(hi1 >= m, one, zero)
        return _dot(i0, ones_col) + _dot(i1, ones_col)

    t16 = search(cnt_hi).astype(jnp.int16)
    strict0 = hi0 > t16
    strict1 = hi1 > t16
    buck0 = hi0 == t16
    buck1 = hi1 == t16
    c_hi = (_dot(jnp.where(strict0, one, zero), ones_col)
            + _dot(jnp.where(strict1, one, zero), ones_col))

    def cnt_low(m):
        i0 = jnp.where(buck0 & (lw0 >= m), one, zero)
        i1 = jnp.where(buck1 & (lw1 >= m), one, zero)
        return c_hi + _dot(i0, ones_col) + _dot(i1, ones_col)

    tlow = search(cnt_low).astype(jnp.int16)
    m0 = strict0 | (buck0 & (lw0 >= tlow))
    m1 = strict1 | (buck1 & (lw1 >= tlow))
    z = jnp.float32(0.0)
    return jnp.where(m0, h0, z), jnp.where(m1, h1, z)


def _dot(a, b):
    return jnp.dot(a, b, preferred_element_type=jnp.float32)


def _in_maxk_body(x_ref, wt0_ref, wt1_ref, b0_ref, b1_ref, o0_ref, o1_ref):
    h0 = _dot(x_ref[...], wt0_ref[...]) + b0_ref[...]
    h1 = _dot(x_ref[...], wt1_ref[...]) + b1_ref[...]
    o0_ref[...], o1_ref[...] = _maxk_mask2(h0, h1)


def _self_mm_body(m0_ref, m1_ref, wsaa, wsab, wsba, wsbb, b0_ref, b1_ref,
                  o0_ref, o1_ref):
    m0, m1 = m0_ref[...], m1_ref[...]
    o0_ref[...] = _dot(m0, wsaa[...]) + _dot(m1, wsba[...]) + b0_ref[...]
    o1_ref[...] = _dot(m0, wsab[...]) + _dot(m1, wsbb[...]) + b1_ref[...]


def _neigh_halves(hs0, hs1, a0, a1, inv, wn):
    hn0 = a0 * inv
    hn1 = a1 * inv
    h0 = hs0 + _dot(hn0, wn[0][0]) + _dot(hn1, wn[1][0])
    h1 = hs1 + _dot(hn0, wn[0][1]) + _dot(hn1, wn[1][1])
    return h0, h1


def _sage_maxk_body(hs0_ref, hs1_ref, a0_ref, a1_ref, deg_ref,
                    wnaa, wnab, wnba, wnbb, o0_ref, o1_ref):
    inv = jnp.float32(1.0) / jnp.maximum(deg_ref[...], jnp.float32(1.0))
    h0, h1 = _neigh_halves(
        hs0_ref[...], hs1_ref[...], a0_ref[...], a1_ref[...], inv,
        ((wnaa[...], wnab[...]), (wnba[...], wnbb[...])))
    o0_ref[...], o1_ref[...] = _maxk_mask2(h0, h1)


def _sage_out_body(hs0_ref, hs1_ref, a0_ref, a1_ref, deg_ref,
                   wnaa, wnab, wnba, wnbb, woa_ref, wob_ref, bo_ref, o_ref):
    inv = jnp.float32(1.0) / jnp.maximum(deg_ref[...], jnp.float32(1.0))
    h0, h1 = _neigh_halves(
        hs0_ref[...], hs1_ref[...], a0_ref[...], a1_ref[...], inv,
        ((wnaa[...], wnab[...]), (wnba[...], wnbb[...])))
    o_ref[...] = (_dot(h0, woa_ref[...]) + _dot(h1, wob_ref[...])
                  + bo_ref[...])


def _row_spec(rb, w):
    return pl.BlockSpec((rb, w), lambda i: (i, 0))


def _full_spec(shape):
    return pl.BlockSpec(shape, lambda i: (0, 0))


def _half_out(rb, nrows):
    return (
        [jax.ShapeDtypeStruct((nrows, H), jnp.float32)] * 2,
        [_row_spec(rb, H)] * 2,
    )


_in_maxk = pl.pallas_call(
    _in_maxk_body,
    grid=(NPAD // RB,),
    in_specs=[_row_spec(RB, F), _full_spec((F, H)), _full_spec((F, H)),
              _full_spec((1, H)), _full_spec((1, H))],
    out_specs=_half_out(RB, NPAD)[1],
    out_shape=_half_out(RB, NPAD)[0],
)

_QSPECS = [_full_spec((H, H))] * 4

_self_mm = pl.pallas_call(
    _self_mm_body,
    grid=(NPAD // RB,),
    in_specs=([_row_spec(RB, H)] * 2 + _QSPECS + [_full_spec((1, H))] * 2),
    out_specs=_half_out(RB, NPAD)[1],
    out_shape=_half_out(RB, NPAD)[0],
)

_sage_maxk = pl.pallas_call(
    _sage_maxk_body,
    grid=(NPAD // RB,),
    in_specs=([_row_spec(RB, H)] * 4
              + [pl.BlockSpec((RB, 1), lambda i: (i, 0))]
              + _QSPECS),
    out_specs=_half_out(RB, NPAD)[1],
    out_shape=_half_out(RB, NPAD)[0],
)

_RB_OUT = 2000

_sage_out = pl.pallas_call(
    _sage_out_body,
    grid=(N // _RB_OUT,),
    in_specs=([_row_spec(_RB_OUT, H)] * 4
              + [pl.BlockSpec((_RB_OUT, 1), lambda i: (i, 0))]
              + _QSPECS
              + [_full_spec((H, F))] * 2 + [_full_spec((1, F))]),
    out_specs=_row_spec(_RB_OUT, F),
    out_shape=jax.ShapeDtypeStruct((N, F), jnp.float32),
)


def _quarters(w):
    return (w[:H, :H], w[:H, H:], w[H:, :H], w[H:, H:])


def _make_sc_agg(with_deg):
    mesh = plsc.VectorSubcoreMesh(core_axis_name="c", subcore_axis_name="s")
    out_types = [jax.ShapeDtypeStruct((NC, NPAD, H), jnp.float32)]
    scratch = [
        pltpu.VMEM((TCHUNKS, BATCH), jnp.int32),
        pltpu.VMEM((TCHUNKS, BATCH), jnp.int32),
        pltpu.VMEM((NBUF, BATCH, H), jnp.float32),
        pltpu.VMEM_SHARED((NPAD, H), jnp.float32),
    ]
    scratch += [pltpu.SemaphoreType.DMA] * (2 * NBUF)
    if with_deg:
        out_types.append(jax.ShapeDtypeStruct((NC, NPAD), jnp.float32))
        scratch += [
            pltpu.VMEM((BATCH,), jnp.float32),
            pltpu.VMEM_SHARED((NPAD,), jnp.float32),
            pltpu.SemaphoreType.DMA,
        ]

    def body(mh0_hbm, mh1_hbm, src_hbm, dst_hbm, sp_hbm, dp_hbm,
             z2_hbm, z1_hbm, *rest):
        if with_deg:
            agg_out, deg_out = rest[0], rest[1]
            rest = rest[2:]
        else:
            agg_out = rest[0]
            rest = rest[1:]
        src_v, dst_v, rows_v, agg_sh = rest[0], rest[1], rest[2], rest[3]
        gsem = rest[4:4 + NBUF]
        ssem = rest[4 + NBUF:4 + 2 * NBUF]
        if with_deg:
            ones_v, deg_sh, dsem = rest[4 + 2 * NBUF:]
        cid = lax.axis_index("c")
        sid = lax.axis_index("s")
        row0 = sid * ROWS_PER_SUB

        pltpu.sync_copy(z2_hbm.at[pl.ds(row0, ROWS_PER_SUB)],
                        agg_sh.at[pl.ds(row0, ROWS_PER_SUB)])
        if with_deg:
            pltpu.sync_copy(z1_hbm.at[pl.ds(row0, ROWS_PER_SUB)],
                            deg_sh.at[pl.ds(row0, ROWS_PER_SUB)])

            def fill(i, c):
                ones_v[pl.ds(i * LANES, LANES)] = jnp.full((LANES,), 1.0,
                                                           jnp.float32)
                return c
            lax.fori_loop(0, BATCH // LANES, fill, 0)

        @pl.when(sid < NS - 1)
        def _():
            pltpu.sync_copy(src_hbm.at[pl.ds(sid * TCHUNKS, TCHUNKS)], src_v)
            pltpu.sync_copy(dst_hbm.at[pl.ds(sid * TCHUNKS, TCHUNKS)], dst_v)

        @pl.when(sid == NS - 1)
        def _():
            pltpu.sync_copy(src_hbm.at[pl.ds((NS - 1) * TCHUNKS, LAST_REAL)],
                            src_v.at[pl.ds(0, LAST_REAL)])
            pltpu.sync_copy(sp_hbm, src_v.at[pl.ds(LAST_REAL, PAD_ROWS)])
            pltpu.sync_copy(dst_hbm.at[pl.ds((NS - 1) * TCHUNKS, LAST_REAL)],
                            dst_v.at[pl.ds(0, LAST_REAL)])
            pltpu.sync_copy(dp_hbm, dst_v.at[pl.ds(LAST_REAL, PAD_ROWS)])
        plsc.subcore_barrier()

        if with_deg:
            def deg_fire(g, c):
                pltpu.async_copy(ones_v, deg_sh.at[dst_v.at[g]], dsem,
                                 add=True)
                return c
            lax.fori_loop(0, TCHUNKS, deg_fire, 0)

        def run(table_hbm):
            def g_start(p, bp):
                pltpu.async_copy(table_hbm.at[src_v.at[p]], rows_v.at[bp],
                                 gsem[bp])

            def g_wait(g, b):
                pltpu.make_async_copy(table_hbm.at[src_v.at[g]], rows_v.at[b],
                                      gsem[b]).wait()

            def s_start(g, b):
                pltpu.async_copy(rows_v.at[b], agg_sh.at[dst_v.at[g]],
                                 ssem[b], add=True)

            def s_wait(g, b):
                pltpu.make_async_copy(rows_v.at[b], agg_sh.at[dst_v.at[g]],
                                      ssem[b]).wait()

            for b in range(PDIST):
                g_start(b, b)

            def outer(t, c):
                g0 = t * NBUF
                for b in range(NBUF):
                    g = g0 + b
                    g_wait(g, b)
                    s_start(g, b)
                    p = g + PDIST
                    bp = (b + PDIST) % NBUF

                    @pl.when(jnp.logical_and(p >= NBUF, p < TCHUNKS))
                    def _():
                        s_wait(p - NBUF, bp)

                    @pl.when(p < TCHUNKS)
                    def _():
                        g_start(p, bp)
                return c
            lax.fori_loop(0, OUTER, outer, 0)

            for b in range(NBUF):
                s_wait(TCHUNKS - NBUF + b, b)

        @pl.when(cid == 0)
        def _():
            run(mh0_hbm)

        @pl.when(cid == 1)
        def _():
            run(mh1_hbm)

        if with_deg:
            def deg_drain(g, c):
                pltpu.make_async_copy(ones_v, deg_sh.at[dst_v.at[g]],
                                      dsem).wait()
                return c
            lax.fori_loop(0, TCHUNKS, deg_drain, 0)

        plsc.subcore_barrier()
        pltpu.sync_copy(agg_sh.at[pl.ds(row0, ROWS_PER_SUB)],
                        agg_out.at[cid, pl.ds(row0, ROWS_PER_SUB)])
        if with_deg:
            pltpu.sync_copy(deg_sh.at[pl.ds(row0, ROWS_PER_SUB)],
                            deg_out.at[cid, pl.ds(row0, ROWS_PER_SUB)])

    return pl.kernel(body, out_type=tuple(out_types), mesh=mesh,
                     scratch_types=scratch,
                     compiler_params=pltpu.CompilerParams(
                         use_tc_tiling_on_sc=False))


@functools.lru_cache(maxsize=None)
def _get_sc_agg(with_deg):
    return _make_sc_agg(with_deg)


def kernel(x, edge_index, W_in, b_in, W_self_0, W_neigh_0, bias_0,
           W_self_1, W_neigh_1, bias_1, W_out, b_out):
    src_p = edge_index[0].reshape(EREAL, BATCH)
    dst_p = edge_index[1].reshape(EREAL, BATCH)
    sp = jnp.zeros((PAD_ROWS, BATCH), jnp.int32)
    dp = jnp.full((PAD_ROWS, BATCH), NPAD - 1, jnp.int32)
    x_p = jnp.concatenate([x, jnp.zeros((NPAD - N, F), jnp.float32)], axis=0)
    z2 = jnp.zeros((NPAD, H), jnp.float32)
    z1 = jnp.zeros((NPAD,), jnp.float32)

    wt_in = W_in.T
    ws0, wn0, ws1, wn1 = W_self_0.T, W_neigh_0.T, W_self_1.T, W_neigh_1.T

    m0a, m0b = _in_maxk(x_p, wt_in[:, :H], wt_in[:, H:],
                        b_in[:H].reshape(1, H), b_in[H:].reshape(1, H))
    (agg0, deg) = _get_sc_agg(True)(m0a, m0b, src_p, dst_p, sp, dp, z2, z1)
    degs = deg[0].reshape(NPAD, 1)
    hs0a, hs0b = _self_mm(m0a, m0b, *_quarters(ws0),
                          bias_0[:H].reshape(1, H), bias_0[H:].reshape(1, H))
    m1a, m1b = _sage_maxk(hs0a, hs0b, agg0[0], agg0[1], degs,
                          *_quarters(wn0))
    (agg1,) = _get_sc_agg(False)(m1a, m1b, src_p, dst_p, sp, dp, z2, z1)
    hs1a, hs1b = _self_mm(m1a, m1b, *_quarters(ws1),
                          bias_1[:H].reshape(1, H), bias_1[H:].reshape(1, H))
    out = _sage_out(hs1a, hs1b, agg1[0], agg1[1], degs,
                    *_quarters(wn1),
                    W_out.T[:H], W_out.T[H:], b_out.reshape(1, F))
    return out


# speedup vs baseline: 1.0247x; 1.0040x over previous
import functools

import jax
import jax.numpy as jnp
from jax import lax
from jax.experimental import pallas as pl
from jax.experimental.pallas import tpu as pltpu
from jax.experimental.pallas import tpu_sc as plsc

N = 10000
E = 320000
F = 128
H = 64
K = 32

NC = 2
NS = 16
LANES = 16

NPAD = 10240
ROWS_PER_SUB = NPAD // NS
BATCH = 128
EPAD = 327680
TCHUNKS = EPAD // (NS * BATCH)
EREAL = E // BATCH
LAST_REAL = EREAL - (NS - 1) * TCHUNKS
PAD_ROWS = TCHUNKS - LAST_REAL

NBUF = 5
PDIST = 4
OUTER = TCHUNKS // NBUF

RB = 5120


def _maxk_mask2(h0, h1):
    b0 = lax.bitcast_convert_type(h0, jnp.int32)
    b1 = lax.bitcast_convert_type(h1, jnp.int32)
    k0 = jnp.where(b0 >= 0, b0, b0 ^ jnp.int32(0x7FFFFFFF))
    k1 = jnp.where(b1 >= 0, b1, b1 ^ jnp.int32(0x7FFFFFFF))
    hi0 = (k0 >> 16).astype(jnp.int16)
    hi1 = (k1 >> 16).astype(jnp.int16)
    lw0 = ((k0 & jnp.int32(0xFFFF)) ^ jnp.int32(0x8000)).astype(jnp.int16)
    lw1 = ((k1 & jnp.int32(0xFFFF)) ^ jnp.int32(0x8000)).astype(jnp.int16)

    one = jnp.bfloat16(1.0)
    zero = jnp.bfloat16(0.0)
    ones_col = jnp.full((H, 1), 1.0, jnp.bfloat16)
    kkf = jnp.float32(K)
    zcol = jnp.sum(jnp.zeros_like(h0), axis=-1, keepdims=True).astype(
        jnp.int32)

    def search(count_fn):
        lo = zcol + jnp.int32(-32768)
        hi = zcol + jnp.int32(32768)

        def body(_, carry):
            lo, hi = carry
            mid = (lo + hi) >> 1
            p = count_fn(mid.astype(jnp.int16)) >= kkf
            return jnp.where(p, mid, lo), jnp.where(p, hi, mid)

        lo, hi = lax.fori_loop(0, 16, body, (lo, hi))
        return lo

    def cnt_hi(m):
        i0 = jnp.where(hi0 >= m, one, zero)
        i1 = jnp.where(hi1 >= m, one, zero)
        return _dot(i0, ones_col) + _dot(i1, ones_col)

    t16 = search(cnt_hi).astype(jnp.int16)
    strict0 = hi0 > t16
    strict1 = hi1 > t16
    buck0 = hi0 == t16
    buck1 = hi1 == t16
    c_hi = (_dot(jnp.where(strict0, one, zero), ones_col)
            + _dot(jnp.where(strict1, one, zero), ones_col))

    def cnt_low(m):
        i0 = jnp.where(buck0 & (lw0 >= m), one, zero)
        i1 = jnp.where(buck1 & (lw1 >= m), one, zero)
        return c_hi + _dot(i0, ones_col) + _dot(i1, ones_col)

    tlow = search(cnt_low).astype(jnp.int16)
    m0 = strict0 | (buck0 & (lw0 >= tlow))
    m1 = strict1 | (buck1 & (lw1 >= tlow))
    z = jnp.float32(0.0)
    return jnp.where(m0, h0, z), jnp.where(m1, h1, z)


def _dot(a, b):
    return jnp.dot(a, b, preferred_element_type=jnp.float32)


def _in_maxk_body(x_ref, wt0_ref, wt1_ref, b0_ref, b1_ref, o0_ref, o1_ref):
    h0 = _dot(x_ref[...], wt0_ref[...]) + b0_ref[...]
    h1 = _dot(x_ref[...], wt1_ref[...]) + b1_ref[...]
    o0_ref[...], o1_ref[...] = _maxk_mask2(h0, h1)


def _self_mm_body(m0_ref, m1_ref, wsaa, wsab, wsba, wsbb, b0_ref, b1_ref,
                  o0_ref, o1_ref):
    m0, m1 = m0_ref[...], m1_ref[...]
    o0_ref[...] = _dot(m0, wsaa[...]) + _dot(m1, wsba[...]) + b0_ref[...]
    o1_ref[...] = _dot(m0, wsab[...]) + _dot(m1, wsbb[...]) + b1_ref[...]


def _neigh_halves(hs0, hs1, a0, a1, inv, wn):
    hn0 = a0 * inv
    hn1 = a1 * inv
    h0 = hs0 + _dot(hn0, wn[0][0]) + _dot(hn1, wn[1][0])
    h1 = hs1 + _dot(hn0, wn[0][1]) + _dot(hn1, wn[1][1])
    return h0, h1


def _sage_maxk_body(hs0_ref, hs1_ref, a0_ref, a1_ref, deg_ref,
                    wnaa, wnab, wnba, wnbb, o0_ref, o1_ref):
    inv = jnp.float32(1.0) / jnp.maximum(deg_ref[...], jnp.float32(1.0))
    h0, h1 = _neigh_halves(
        hs0_ref[...], hs1_ref[...], a0_ref[...], a1_ref[...], inv,
        ((wnaa[...], wnab[...]), (wnba[...], wnbb[...])))
    o0_ref[...], o1_ref[...] = _maxk_mask2(h0, h1)


def _sage_out_body(hs0_ref, hs1_ref, a0_ref, a1_ref, deg_ref,
                   wnaa, wnab, wnba, wnbb, woa_ref, wob_ref, bo_ref, o_ref):
    inv = jnp.float32(1.0) / jnp.maximum(deg_ref[...], jnp.float32(1.0))
    h0, h1 = _neigh_halves(
        hs0_ref[...], hs1_ref[...], a0_ref[...], a1_ref[...], inv,
        ((wnaa[...], wnab[...]), (wnba[...], wnbb[...])))
    o_ref[...] = (_dot(h0, woa_ref[...]) + _dot(h1, wob_ref[...])
                  + bo_ref[...])


def _row_spec(rb, w):
    return pl.BlockSpec((rb, w), lambda i: (i, 0))


def _full_spec(shape):
    return pl.BlockSpec(shape, lambda i: (0, 0))


def _half_out(rb, nrows):
    return (
        [jax.ShapeDtypeStruct((nrows, H), jnp.float32)] * 2,
        [_row_spec(rb, H)] * 2,
    )


_in_maxk = pl.pallas_call(
    _in_maxk_body,
    grid=(NPAD // RB,),
    in_specs=[_row_spec(RB, F), _full_spec((F, H)), _full_spec((F, H)),
              _full_spec((1, H)), _full_spec((1, H))],
    out_specs=_half_out(RB, NPAD)[1],
    out_shape=_half_out(RB, NPAD)[0],
)

_QSPECS = [_full_spec((H, H))] * 4

_self_mm = pl.pallas_call(
    _self_mm_body,
    grid=(NPAD // RB,),
    in_specs=([_row_spec(RB, H)] * 2 + _QSPECS + [_full_spec((1, H))] * 2),
    out_specs=_half_out(RB, NPAD)[1],
    out_shape=_half_out(RB, NPAD)[0],
)

_sage_maxk = pl.pallas_call(
    _sage_maxk_body,
    grid=(NPAD // RB,),
    in_specs=([_row_spec(RB, H)] * 4
              + [pl.BlockSpec((RB, 1), lambda i: (i, 0))]
              + _QSPECS),
    out_specs=_half_out(RB, NPAD)[1],
    out_shape=_half_out(RB, NPAD)[0],
)

_RB_OUT = 2000

_sage_out = pl.pallas_call(
    _sage_out_body,
    grid=(N // _RB_OUT,),
    in_specs=([_row_spec(_RB_OUT, H)] * 4
              + [pl.BlockSpec((_RB_OUT, 1), lambda i: (i, 0))]
              + _QSPECS
              + [_full_spec((H, F))] * 2 + [_full_spec((1, F))]),
    out_specs=_row_spec(_RB_OUT, F),
    out_shape=jax.ShapeDtypeStruct((N, F), jnp.float32),
)


def _quarters(w):
    return (w[:H, :H], w[:H, H:], w[H:, :H], w[H:, H:])


def _make_sc_agg(with_deg):
    mesh = plsc.VectorSubcoreMesh(core_axis_name="c", subcore_axis_name="s")
    out_types = [jax.ShapeDtypeStruct((NC, NPAD, H), jnp.float32)]
    scratch = [
        pltpu.VMEM((TCHUNKS, BATCH), jnp.int32),
        pltpu.VMEM((TCHUNKS, BATCH), jnp.int32),
        pltpu.VMEM((NBUF, BATCH, H), jnp.float32),
        pltpu.VMEM_SHARED((NPAD, H), jnp.float32),
    ]
    scratch += [pltpu.SemaphoreType.DMA] * (2 * NBUF)
    if with_deg:
        out_types.append(jax.ShapeDtypeStruct((NC, NPAD), jnp.float32))
        scratch += [
            pltpu.VMEM((BATCH,), jnp.float32),
            pltpu.VMEM_SHARED((NPAD,), jnp.float32),
            pltpu.SemaphoreType.DMA,
        ]

    def body(mh0_hbm, mh1_hbm, src_hbm, dst_hbm, sp_hbm, dp_hbm,
             z2_hbm, z1_hbm, *rest):
        if with_deg:
            agg_out, deg_out = rest[0], rest[1]
            rest = rest[2:]
        else:
            agg_out = rest[0]
            rest = rest[1:]
        src_v, dst_v, rows_v, agg_sh = rest[0], rest[1], rest[2], rest[3]
        gsem = rest[4:4 + NBUF]
        ssem = rest[4 + NBUF:4 + 2 * NBUF]
        if with_deg:
            ones_v, deg_sh, dsem = rest[4 + 2 * NBUF:]
        cid = lax.axis_index("c")
        sid = lax.axis_index("s")
        row0 = sid * ROWS_PER_SUB

        pltpu.sync_copy(z2_hbm.at[pl.ds(row0, ROWS_PER_SUB)],
                        agg_sh.at[pl.ds(row0, ROWS_PER_SUB)])
        if with_deg:
            pltpu.sync_copy(z1_hbm.at[pl.ds(row0, ROWS_PER_SUB)],
                            deg_sh.at[pl.ds(row0, ROWS_PER_SUB)])

            def fill(i, c):
                ones_v[pl.ds(i * LANES, LANES)] = jnp.full((LANES,), 1.0,
                                                           jnp.float32)
                return c
            lax.fori_loop(0, BATCH // LANES, fill, 0)

        @pl.when(sid < NS - 1)
        def _():
            pltpu.sync_copy(src_hbm.at[pl.ds(sid * TCHUNKS, TCHUNKS)], src_v)
            pltpu.sync_copy(dst_hbm.at[pl.ds(sid * TCHUNKS, TCHUNKS)], dst_v)

        @pl.when(sid == NS - 1)
        def _():
            pltpu.sync_copy(src_hbm.at[pl.ds((NS - 1) * TCHUNKS, LAST_REAL)],
                            src_v.at[pl.ds(0, LAST_REAL)])
            pltpu.sync_copy(sp_hbm, src_v.at[pl.ds(LAST_REAL, PAD_ROWS)])
            pltpu.sync_copy(dst_hbm.at[pl.ds((NS - 1) * TCHUNKS, LAST_REAL)],
                            dst_v.at[pl.ds(0, LAST_REAL)])
            pltpu.sync_copy(dp_hbm, dst_v.at[pl.ds(LAST_REAL, PAD_ROWS)])
        plsc.subcore_barrier()

        if with_deg:
            def deg_fire(g, c):
                pltpu.async_copy(ones_v, deg_sh.at[dst_v.at[g]], dsem,
                                 add=True)
                return c
            lax.fori_loop(0, TCHUNKS, deg_fire, 0)

        def run(table_hbm):
            def g_start(p, bp):
                pltpu.async_copy(table_hbm.at[src_v.at[p]], rows_v.at[bp],
                                 gsem[bp])

            def g_wait(g, b):
                pltpu.make_async_copy(table_hbm.at[src_v.at[g]], rows_v.at[b],
                                      gsem[b]).wait()

            def s_start(g, b):
                pltpu.async_copy(rows_v.at[b], agg_sh.at[dst_v.at[g]],
                                 ssem[b], add=True)

            def s_wait(g, b):
                pltpu.make_async_copy(rows_v.at[b], agg_sh.at[dst_v.at[g]],
                                      ssem[b]).wait()

            for b in range(PDIST):
                g_start(b, b)

            def outer(t, c):
                g0 = t * NBUF
                for b in range(NBUF):
                    g = g0 + b
                    g_wait(g, b)
                    s_start(g, b)
                    p = g + PDIST
                    bp = (b + PDIST) % NBUF

                    @pl.when(jnp.logical_and(p >= NBUF, p < TCHUNKS))
                    def _():
                        s_wait(p - NBUF, bp)

                    @pl.when(p < TCHUNKS)
                    def _():
                        g_start(p, bp)
                return c
            lax.fori_loop(0, OUTER, outer, 0)

            for b in range(NBUF):
                s_wait(TCHUNKS - NBUF + b, b)

        @pl.when(cid == 0)
        def _():
            run(mh0_hbm)

        @pl.when(cid == 1)
        def _():
            run(mh1_hbm)

        if with_deg:
            def deg_drain(g, c):
                pltpu.make_async_copy(ones_v, deg_sh.at[dst_v.at[g]],
                                      dsem).wait()
                return c
            lax.fori_loop(0, TCHUNKS, deg_drain, 0)

        plsc.subcore_barrier()
        pltpu.sync_copy(agg_sh.at[pl.ds(row0, ROWS_PER_SUB)],
                        agg_out.at[cid, pl.ds(row0, ROWS_PER_SUB)])
        if with_deg:
            pltpu.sync_copy(deg_sh.at[pl.ds(row0, ROWS_PER_SUB)],
                            deg_out.at[cid, pl.ds(row0, ROWS_PER_SUB)])

    return pl.kernel(body, out_type=tuple(out_types), mesh=mesh,
                     scratch_types=scratch,
                     compiler_params=pltpu.CompilerParams(
                         use_tc_tiling_on_sc=False))


@functools.lru_cache(maxsize=None)
def _get_sc_agg(with_deg):
    return _make_sc_agg(with_deg)


def kernel(x, edge_index, W_in, b_in, W_self_0, W_neigh_0, bias_0,
           W_self_1, W_neigh_1, bias_1, W_out, b_out):
    src_p = edge_index[0].reshape(EREAL, BATCH)
    dst_p = edge_index[1].reshape(EREAL, BATCH)
    sp = jnp.zeros((PAD_ROWS, BATCH), jnp.int32)
    dp = jnp.full((PAD_ROWS, BATCH), NPAD - 1, jnp.int32)
    x_p = jnp.concatenate([x, jnp.zeros((NPAD - N, F), jnp.float32)], axis=0)
    z2 = jnp.zeros((NPAD, H), jnp.float32)
    z1 = jnp.zeros((NPAD,), jnp.float32)

    wt_in = W_in.T
    ws0, wn0, ws1, wn1 = W_self_0.T, W_neigh_0.T, W_self_1.T, W_neigh_1.T

    m0a, m0b = _in_maxk(x_p, wt_in[:, :H], wt_in[:, H:],
                        b_in[:H].reshape(1, H), b_in[H:].reshape(1, H))
    (agg0, deg) = _get_sc_agg(True)(m0a, m0b, src_p, dst_p, sp, dp, z2, z1)
    degs = deg[0].reshape(NPAD, 1)
    hs0a, hs0b = _self_mm(m0a, m0b, *_quarters(ws0),
                          bias_0[:H].reshape(1, H), bias_0[H:].reshape(1, H))
    m1a, m1b = _sage_maxk(hs0a, hs0b, agg0[0], agg0[1], degs,
                          *_quarters(wn0))
    (agg1,) = _get_sc_agg(False)(m1a, m1b, src_p, dst_p, sp, dp, z2, z1)
    hs1a, hs1b = _self_mm(m1a, m1b, *_quarters(ws1),
                          bias_1[:H].reshape(1, H), bias_1[H:].reshape(1, H))
    out = _sage_out(hs1a, hs1b, agg1[0], agg1[1], degs,
                    *_quarters(wn1),
                    W_out.T[:H], W_out.T[H:], b_out.reshape(1, F))
    return out
